# Initial kernel scaffold; baseline (speedup 1.0000x reference)
import jax
import jax.numpy as jnp
from jax.experimental import pallas as pl


def kernel(x, Wq, Wk, Wv, Wi, bi, Wf, bf, Wo, bo, W_out, ln_g, ln_b):
    raise NotImplementedError("write your pallas kernel here")


# speedup vs baseline: 20.3511x; 20.3511x over previous
import math

import jax
import jax.numpy as jnp
from jax.experimental import pallas as pl
from jax.experimental.pallas import tpu as pltpu

L = 128


def _mlstm_chunk_kernel(H, Dh, NC,
                        x_ref, wq_ref, wk_ref, wv_ref, wi_ref, bi_ref,
                        wf_ref, bf_ref, wo_ref, bo_ref, wout_ref, g_ref, be_ref,
                        out_ref, c_out_ref, n_out_ref, c_s, n_s):
    c = pl.program_id(1)

    @pl.when(c == 0)
    def _():
        c_s[...] = jnp.zeros_like(c_s)
        n_s[...] = jnp.zeros_like(n_s)

    xb = x_ref[0]

    def dot_t(a, b):
        return jax.lax.dot_general(a, b, (((1,), (1,)), ((), ())),
                                   preferred_element_type=jnp.float32)

    q = dot_t(xb, wq_ref[...])
    k = dot_t(xb, wk_ref[...]) * (1.0 / math.sqrt(Dh))
    v = dot_t(xb, wv_ref[...])
    a_i = dot_t(xb, wi_ref[...]) + bi_ref[...]
    a_f = dot_t(xb, wf_ref[...]) + bf_ref[...]
    o = jax.nn.sigmoid(dot_t(xb, wo_ref[...]) + bo_ref[...])

    row = jax.lax.broadcasted_iota(jnp.int32, (L, L), 0)
    col = jax.lax.broadcasted_iota(jnp.int32, (L, L), 1)
    tril = col <= row
    tril_f = jnp.where(tril, 1.0, 0.0)
    lf = jax.lax.dot_general(tril_f, a_f, (((1,), (0,)), ((), ())),
                             preferred_element_type=jnp.float32)

    fv = jnp.exp(lf)
    wk_dec = jnp.exp(a_i - lf)

    outs = []
    for h in range(H):
        sl = slice(h * Dh, (h + 1) * Dh)
        qp = q[:, sl] * fv[:, h:h + 1]
        kp = k[:, sl] * wk_dec[:, h:h + 1]
        vh = v[:, sl]
        c_in = c_s[h]
        n_in = n_s[h:h + 1, :]

        s_mat = jnp.where(tril, dot_t(qp, kp), 0.0)
        h_intra = jax.lax.dot_general(s_mat, vh, (((1,), (0,)), ((), ())),
                                      preferred_element_type=jnp.float32)
        h_inter = dot_t(qp, c_in)
        nq = jnp.sum(s_mat, axis=1, keepdims=True) + dot_t(qp, n_in)
        denom = jnp.maximum(jnp.abs(nq), 1.0)
        outs.append((h_intra + h_inter) / denom * o[:, sl])

        f_last = fv[L - 1:L, h:h + 1]
        m_upd = jax.lax.dot_general(vh, kp, (((0,), (0,)), ((), ())),
                                    preferred_element_type=jnp.float32)
        c_s[h] = f_last * (c_in + m_upd)
        n_s[h:h + 1, :] = f_last * (n_in + jnp.sum(kp, axis=0, keepdims=True))

    hs = jnp.concatenate(outs, axis=1)
    mu = jnp.mean(hs, axis=-1, keepdims=True)
    var = jnp.mean((hs - mu) ** 2, axis=-1, keepdims=True)
    hn = (hs - mu) * jax.lax.rsqrt(var + 1e-5) * g_ref[...] + be_ref[...]
    out_ref[0] = dot_t(hn, wout_ref[...])

    @pl.when(c == NC - 1)
    def _():
        c_out_ref[0] = c_s[...]
        n_out_ref[0] = n_s[...]


def kernel(x, Wq, Wk, Wv, Wi, bi, Wf, bf, Wo, bo, W_out, ln_g, ln_b):
    B, T, IN = x.shape
    HD = Wq.shape[0]
    H = Wi.shape[0]
    Dh = HD // H
    HID = W_out.shape[0]
    NC = T // L
    f32 = jnp.float32

    import functools
    body = functools.partial(_mlstm_chunk_kernel, H, Dh, NC)

    full = lambda shape: pl.BlockSpec(shape, lambda b, c: (0,) * len(shape))
    out, C, n = pl.pallas_call(
        body,
        grid=(B, NC),
        in_specs=[
            pl.BlockSpec((1, L, IN), lambda b, c: (b, c, 0)),
            full((HD, IN)), full((HD, IN)), full((HD, IN)),
            full((H, IN)), full((1, H)),
            full((H, IN)), full((1, H)),
            full((HD, IN)), full((1, HD)),
            full((HID, HD)), full((1, HD)), full((1, HD)),
        ],
        out_specs=[
            pl.BlockSpec((1, L, HID), lambda b, c: (b, c, 0)),
            pl.BlockSpec((1, H, Dh, Dh), lambda b, c: (b, 0, 0, 0)),
            pl.BlockSpec((1, H, Dh), lambda b, c: (b, 0, 0)),
        ],
        out_shape=[
            jax.ShapeDtypeStruct((B, T, HID), f32),
            jax.ShapeDtypeStruct((B, H, Dh, Dh), f32),
            jax.ShapeDtypeStruct((B, H, Dh), f32),
        ],
        scratch_shapes=[
            pltpu.VMEM((H, Dh, Dh), f32),
            pltpu.VMEM((H, Dh), f32),
        ],
        compiler_params=pltpu.CompilerParams(
            dimension_semantics=("parallel", "arbitrary"),
            vmem_limit_bytes=48 * 1024 * 1024,
        ),
        name="mlstm_chunk",
    )(x, Wq, Wk, Wv,
      Wi, bi.reshape(1, H), Wf, bf.reshape(1, H),
      Wo, bo.reshape(1, HD), W_out, ln_g.reshape(1, HD), ln_b.reshape(1, HD))
    return out, (C, n)
